# baseline (speedup 1.0000x reference)
import functools

import jax
import jax.numpy as jnp
from jax import lax
from jax.experimental import pallas as pl
from jax.experimental.pallas import tpu as pltpu

_CHUNK = 64


def _cdiv(a, b):
    return (a + b - 1) // b


def _cmp_exchange(v, idx, vp, ip, low, asc):
    less = (v < vp) | ((v == vp) & (idx < ip))
    sel = less == (low == asc)
    return jnp.where(sel, v, vp), jnp.where(sel, idx, ip)


def _stage(v, idx, pos, j, asc, m):
    low = (pos & j) == 0
    vp = jnp.where(low, pltpu.roll(v, m - j, 0), pltpu.roll(v, j, 0))
    ip = jnp.where(low, pltpu.roll(idx, m - j, 0), pltpu.roll(idx, j, 0))
    return _cmp_exchange(v, idx, vp, ip, low, asc)


def _sort_topk_body(d_ref, t_ref, gt_ref, ki_ref, kr_ref, kv_ref,
                    vs_ref, is_ref, *, n, kpad):
    b = d_ref.shape[0]
    c = min(_CHUNK, n)
    nch = n // c
    pos_c = lax.broadcasted_iota(jnp.int32, (c, 1), 0)

    d0 = d_ref[...]
    gt_ref[...] = jnp.broadcast_to(jnp.sum(d0 * t_ref[...]), (1, 1, 1))
    vs_ref[...] = d0.T

    def pass0(ci, _):
        base = ci * c
        v = vs_ref[pl.ds(base, c), :]
        idx = lax.broadcasted_iota(jnp.int32, (c, b), 0) + base
        pos = pos_c + base
        size = 2
        while size <= c:
            asc = (pos & size) == 0
            j = size // 2
            while j >= 1:
                v, idx = _stage(v, idx, pos_c, j, asc, c)
                j //= 2
            size *= 2
        vs_ref[pl.ds(base, c), :] = v
        is_ref[pl.ds(base, c), :] = idx
        return 0

    lax.fori_loop(0, nch, pass0, 0)

    pos_f = lax.broadcasted_iota(jnp.int32, (n, 1), 0)
    size = 2 * c
    while size <= n:
        j = size // 2
        while j >= c:
            asc = (pos_f & size) == 0
            v = vs_ref[...]
            idx = is_ref[...]
            v, idx = _stage(v, idx, pos_f, j, asc, n)
            vs_ref[...] = v
            is_ref[...] = idx
            j //= 2

        def passf(ci, _, size=size):
            base = ci * c
            v = vs_ref[pl.ds(base, c), :]
            idx = is_ref[pl.ds(base, c), :]
            asc = ((pos_c + base) & size) == 0
            j = c // 2
            while j >= 1:
                v, idx = _stage(v, idx, pos_c, j, asc, c)
                j //= 2
            vs_ref[pl.ds(base, c), :] = v
            is_ref[pl.ds(base, c), :] = idx
            return 0

        lax.fori_loop(0, nch, passf, 0)
        size *= 2

    b2 = ki_ref.shape[0]
    k = ki_ref.shape[1] // 2
    r_total = pl.num_programs(0) * b2
    ki = is_ref[pl.ds(0, kpad), :].T[:, :k]
    kv = vs_ref[pl.ds(0, kpad), :].T[:, :k]
    rows = (lax.broadcasted_iota(jnp.int32, (b2, k), 0)
            + pl.program_id(0) * b2)
    dst = ki + r_total
    ki_ref[...] = jnp.stack([rows, dst], axis=2).reshape(b2, 2 * k)
    kr_ref[...] = jnp.stack([dst, rows], axis=2).reshape(b2, 2 * k)
    kv_ref[...] = jnp.stack([kv, kv], axis=2).reshape(b2, 2 * k)


def _topk_call(d, t):
    r, n = d.shape
    k = min(r, 10 + 2 * (r // 10))
    kpad = min(_cdiv(k, 8) * 8, n)
    b = min(64, r)
    g = r // b
    gt_p, e0, e1, ea = pl.pallas_call(
        functools.partial(_sort_topk_body, n=n, kpad=kpad),
        grid=(g,),
        in_specs=[
            pl.BlockSpec((b, n), lambda i: (i, 0)),
            pl.BlockSpec((b, n), lambda i: (i, 0)),
        ],
        out_specs=[
            pl.BlockSpec((1, 1, 1), lambda i: (i, 0, 0)),
            pl.BlockSpec((b, 2 * k), lambda i: (i, 0)),
            pl.BlockSpec((b, 2 * k), lambda i: (i, 0)),
            pl.BlockSpec((b, 2 * k), lambda i: (i, 0)),
        ],
        out_shape=[
            jax.ShapeDtypeStruct((g, 1, 1), jnp.float32),
            jax.ShapeDtypeStruct((r, 2 * k), jnp.int32),
            jax.ShapeDtypeStruct((r, 2 * k), jnp.int32),
            jax.ShapeDtypeStruct((r, 2 * k), jnp.float32),
        ],
        scratch_shapes=[
            pltpu.VMEM((n, b), jnp.float32),
            pltpu.VMEM((n, b), jnp.int32),
        ],
        compiler_params=pltpu.CompilerParams(
            dimension_semantics=("parallel",),
            vmem_limit_bytes=67_000_000,
        ),
    )(d, t)
    return gt_p, e0, e1, ea


def kernel(distance_matrix, target):
    r, n = distance_matrix.shape

    gt_p, e0, e1, ea = _topk_call(distance_matrix, target)
    gt = jnp.sum(gt_p)

    edge_index = jnp.stack([e0.reshape(-1), e1.reshape(-1)], axis=0)
    edge_attr = ea.reshape(-1, 1)

    x = jnp.zeros((r + n, 8), dtype=jnp.float32)
    y = target.reshape(-1, 1)
    cost_vec = distance_matrix.reshape(-1, 1)
    return (gt, x, edge_index, edge_attr, y, cost_vec)


# speedup vs baseline: 1.2003x; 1.2003x over previous
import functools

import jax
import jax.numpy as jnp
from jax import lax
from jax.experimental import pallas as pl
from jax.experimental.pallas import tpu as pltpu

_CHUNK = 64


def _cdiv(a, b):
    return (a + b - 1) // b


def _cmp_exchange(v, idx, vp, ip, low, asc):
    less = (v < vp) | ((v == vp) & (idx < ip))
    sel = less == (low == asc)
    return jnp.where(sel, v, vp), jnp.where(sel, idx, ip)


def _stage(v, idx, pos, j, asc, m):
    low = (pos & j) == 0
    vp = jnp.where(low, pltpu.roll(v, m - j, 0), pltpu.roll(v, j, 0))
    ip = jnp.where(low, pltpu.roll(idx, m - j, 0), pltpu.roll(idx, j, 0))
    return _cmp_exchange(v, idx, vp, ip, low, asc)


def _sort_topk_body(d_ref, t_ref, gt_ref, ki_ref, kr_ref, kv_ref,
                    vs_ref, is_ref, *, n, kpad):
    b = d_ref.shape[0]
    c = min(_CHUNK, n)
    nch = n // c
    pos_c = lax.broadcasted_iota(jnp.int32, (c, 1), 0)

    d0 = d_ref[...]
    gt_ref[...] = jnp.broadcast_to(jnp.sum(d0 * t_ref[...]), (1, 1, 1))
    vs_ref[...] = d0.T

    def pass0(ci, _):
        base = ci * c
        v = vs_ref[pl.ds(base, c), :]
        idx = lax.broadcasted_iota(jnp.int32, (c, b), 0) + base
        pos = pos_c + base
        size = 2
        while size <= c:
            asc = (pos & size) == 0
            j = size // 2
            while j >= 1:
                v, idx = _stage(v, idx, pos_c, j, asc, c)
                j //= 2
            size *= 2
        vs_ref[pl.ds(base, c), :] = v
        is_ref[pl.ds(base, c), :] = idx
        return 0

    lax.fori_loop(0, nch, pass0, 0)

    pos_f = lax.broadcasted_iota(jnp.int32, (n, 1), 0)
    size = 2 * c
    while size <= n:
        asc_f = (pos_f & size) == 0
        n_big = (size // 2).bit_length() - c.bit_length() + 1

        def big_pass(ti, _, size=size, asc_f=asc_f):
            j = jnp.int32(size // 2) >> ti
            v = vs_ref[...]
            idx = is_ref[...]
            low = (pos_f & j) == 0
            vp = jnp.where(low, pltpu.roll(v, n - j, 0), pltpu.roll(v, j, 0))
            ip = jnp.where(low, pltpu.roll(idx, n - j, 0),
                           pltpu.roll(idx, j, 0))
            v, idx = _cmp_exchange(v, idx, vp, ip, low, asc_f)
            vs_ref[...] = v
            is_ref[...] = idx
            return 0

        lax.fori_loop(0, n_big, big_pass, 0)

        def passf(ci, _, size=size):
            base = ci * c
            v = vs_ref[pl.ds(base, c), :]
            idx = is_ref[pl.ds(base, c), :]
            asc = ((pos_c + base) & size) == 0
            j = c // 2
            while j >= 1:
                v, idx = _stage(v, idx, pos_c, j, asc, c)
                j //= 2
            vs_ref[pl.ds(base, c), :] = v
            is_ref[pl.ds(base, c), :] = idx
            return 0

        lax.fori_loop(0, nch, passf, 0)
        size *= 2

    b2 = ki_ref.shape[0]
    k = ki_ref.shape[1] // 2
    r_total = pl.num_programs(0) * b2
    ki_t = is_ref[pl.ds(0, kpad), :][:k]
    kv_t = vs_ref[pl.ds(0, kpad), :][:k]
    rows_t = (lax.broadcasted_iota(jnp.int32, (k, b2), 1)
              + pl.program_id(0) * b2)
    dst_t = ki_t + r_total
    ki_ref[...] = jnp.stack([rows_t, dst_t], axis=1).reshape(2 * k, b2).T
    kr_ref[...] = jnp.stack([dst_t, rows_t], axis=1).reshape(2 * k, b2).T
    kv_ref[...] = jnp.stack([kv_t, kv_t], axis=1).reshape(2 * k, b2).T


def _topk_call(d, t):
    r, n = d.shape
    k = min(r, 10 + 2 * (r // 10))
    kpad = min(_cdiv(k, 8) * 8, n)
    b = min(128, r)
    g = r // b
    gt_p, e0, e1, ea = pl.pallas_call(
        functools.partial(_sort_topk_body, n=n, kpad=kpad),
        grid=(g,),
        in_specs=[
            pl.BlockSpec((b, n), lambda i: (i, 0)),
            pl.BlockSpec((b, n), lambda i: (i, 0)),
        ],
        out_specs=[
            pl.BlockSpec((1, 1, 1), lambda i: (i, 0, 0)),
            pl.BlockSpec((b, 2 * k), lambda i: (i, 0)),
            pl.BlockSpec((b, 2 * k), lambda i: (i, 0)),
            pl.BlockSpec((b, 2 * k), lambda i: (i, 0)),
        ],
        out_shape=[
            jax.ShapeDtypeStruct((g, 1, 1), jnp.float32),
            jax.ShapeDtypeStruct((r, 2 * k), jnp.int32),
            jax.ShapeDtypeStruct((r, 2 * k), jnp.int32),
            jax.ShapeDtypeStruct((r, 2 * k), jnp.float32),
        ],
        scratch_shapes=[
            pltpu.VMEM((n, b), jnp.float32),
            pltpu.VMEM((n, b), jnp.int32),
        ],
        compiler_params=pltpu.CompilerParams(
            dimension_semantics=("parallel",),
            vmem_limit_bytes=67_000_000,
        ),
    )(d, t)
    return gt_p, e0, e1, ea


def kernel(distance_matrix, target):
    r, n = distance_matrix.shape

    gt_p, e0, e1, ea = _topk_call(distance_matrix, target)
    gt = jnp.sum(gt_p)

    edge_index = jnp.stack([e0.reshape(-1), e1.reshape(-1)], axis=0)
    edge_attr = ea.reshape(-1, 1)

    x = jnp.zeros((r + n, 8), dtype=jnp.float32)
    y = target.reshape(-1, 1)
    cost_vec = distance_matrix.reshape(-1, 1)
    return (gt, x, edge_index, edge_attr, y, cost_vec)


# speedup vs baseline: 2.6425x; 2.2015x over previous
import functools

import jax
import jax.numpy as jnp
from jax import lax
from jax.experimental import pallas as pl
from jax.experimental.pallas import tpu as pltpu

_CHUNK = 64


def _cdiv(a, b):
    return (a + b - 1) // b


def _cmp_exchange(v, idx, vp, ip, low, asc):
    less = (v < vp) | ((v == vp) & (idx < ip))
    sel = less == (low == asc)
    return jnp.where(sel, v, vp), jnp.where(sel, idx, ip)


def _stage(v, idx, pos, j, asc, m):
    low = (pos & j) == 0
    vp = jnp.where(low, pltpu.roll(v, m - j, 0), pltpu.roll(v, j, 0))
    ip = jnp.where(low, pltpu.roll(idx, m - j, 0), pltpu.roll(idx, j, 0))
    return _cmp_exchange(v, idx, vp, ip, low, asc)


def _sort_topk_body(d_ref, t_ref, gt_ref, ki_ref, kr_ref, kv_ref,
                    vs_ref, is_ref, *, n, kpad):
    b = d_ref.shape[0]
    c = min(_CHUNK, n)
    nch = n // c
    pos_c = lax.broadcasted_iota(jnp.int32, (c, 1), 0)

    d0 = d_ref[...]
    gt_ref[...] = jnp.broadcast_to(jnp.sum(d0 * t_ref[...]), (1, 1, 1))
    vs_ref[...] = d0.T

    def pass0(ci, _):
        base = ci * c
        v = vs_ref[pl.ds(base, c), :]
        idx = lax.broadcasted_iota(jnp.int32, (c, b), 0) + base
        pos = pos_c + base
        size = 2
        while size <= c:
            asc = (pos & size) == 0
            j = size // 2
            while j >= 1:
                v, idx = _stage(v, idx, pos_c, j, asc, c)
                j //= 2
            size *= 2
        vs_ref[pl.ds(base, c), :] = v
        is_ref[pl.ds(base, c), :] = idx
        return 0

    lax.fori_loop(0, nch, pass0, 0)

    pos_f = lax.broadcasted_iota(jnp.int32, (n, 1), 0)
    size = 2 * c
    while size <= n:
        asc_f = (pos_f & size) == 0
        j = size // 2
        while j >= c:
            v = vs_ref[...]
            idx = is_ref[...]
            v, idx = _stage(v, idx, pos_f, j, asc_f, n)
            vs_ref[...] = v
            is_ref[...] = idx
            j //= 2

        def passf(ci, _, size=size):
            base = ci * c
            v = vs_ref[pl.ds(base, c), :]
            idx = is_ref[pl.ds(base, c), :]
            asc = ((pos_c + base) & size) == 0
            j = c // 2
            while j >= 1:
                v, idx = _stage(v, idx, pos_c, j, asc, c)
                j //= 2
            vs_ref[pl.ds(base, c), :] = v
            is_ref[pl.ds(base, c), :] = idx
            return 0

        lax.fori_loop(0, nch, passf, 0)
        size *= 2

    b2 = ki_ref.shape[0]
    k = ki_ref.shape[1] // 2
    r_total = pl.num_programs(0) * b2
    ki_t = is_ref[pl.ds(0, kpad), :][:k]
    kv_t = vs_ref[pl.ds(0, kpad), :][:k]
    rows_t = (lax.broadcasted_iota(jnp.int32, (k, b2), 1)
              + pl.program_id(0) * b2)
    dst_t = ki_t + r_total
    ki_ref[...] = jnp.stack([rows_t, dst_t], axis=1).reshape(2 * k, b2).T
    kr_ref[...] = jnp.stack([dst_t, rows_t], axis=1).reshape(2 * k, b2).T
    kv_ref[...] = jnp.stack([kv_t, kv_t], axis=1).reshape(2 * k, b2).T


def _topk_call(d, t):
    r, n = d.shape
    k = min(r, 10 + 2 * (r // 10))
    kpad = min(_cdiv(k, 8) * 8, n)
    b = min(128, r)
    g = r // b
    gt_p, e0, e1, ea = pl.pallas_call(
        functools.partial(_sort_topk_body, n=n, kpad=kpad),
        grid=(g,),
        in_specs=[
            pl.BlockSpec((b, n), lambda i: (i, 0)),
            pl.BlockSpec((b, n), lambda i: (i, 0)),
        ],
        out_specs=[
            pl.BlockSpec((1, 1, 1), lambda i: (i, 0, 0)),
            pl.BlockSpec((b, 2 * k), lambda i: (i, 0)),
            pl.BlockSpec((b, 2 * k), lambda i: (i, 0)),
            pl.BlockSpec((b, 2 * k), lambda i: (i, 0)),
        ],
        out_shape=[
            jax.ShapeDtypeStruct((g, 1, 1), jnp.float32),
            jax.ShapeDtypeStruct((r, 2 * k), jnp.int32),
            jax.ShapeDtypeStruct((r, 2 * k), jnp.int32),
            jax.ShapeDtypeStruct((r, 2 * k), jnp.float32),
        ],
        scratch_shapes=[
            pltpu.VMEM((n, b), jnp.float32),
            pltpu.VMEM((n, b), jnp.int32),
        ],
        compiler_params=pltpu.CompilerParams(
            dimension_semantics=("parallel",),
            vmem_limit_bytes=67_000_000,
        ),
    )(d, t)
    return gt_p, e0, e1, ea


def kernel(distance_matrix, target):
    r, n = distance_matrix.shape

    gt_p, e0, e1, ea = _topk_call(distance_matrix, target)
    gt = jnp.sum(gt_p)

    edge_index = jnp.stack([e0.reshape(-1), e1.reshape(-1)], axis=0)
    edge_attr = ea.reshape(-1, 1)

    x = jnp.zeros((r + n, 8), dtype=jnp.float32)
    y = target.reshape(-1, 1)
    cost_vec = distance_matrix.reshape(-1, 1)
    return (gt, x, edge_index, edge_attr, y, cost_vec)


# speedup vs baseline: 2.6432x; 1.0003x over previous
import functools

import jax
import jax.numpy as jnp
from jax import lax
from jax.experimental import pallas as pl
from jax.experimental.pallas import tpu as pltpu

_CHUNK = 128


def _cdiv(a, b):
    return (a + b - 1) // b


def _cmp_exchange(v, idx, vp, ip, low, asc):
    less = (v < vp) | ((v == vp) & (idx < ip))
    sel = less == (low == asc)
    return jnp.where(sel, v, vp), jnp.where(sel, idx, ip)


def _stage(v, idx, pos, j, asc, m):
    low = (pos & j) == 0
    vp = jnp.where(low, pltpu.roll(v, m - j, 0), pltpu.roll(v, j, 0))
    ip = jnp.where(low, pltpu.roll(idx, m - j, 0), pltpu.roll(idx, j, 0))
    return _cmp_exchange(v, idx, vp, ip, low, asc)


def _sort_topk_body(d_ref, t_ref, gt_ref, ki_ref, kr_ref, kv_ref,
                    vs_ref, is_ref, *, n, kpad):
    b = d_ref.shape[0]
    c = min(_CHUNK, n)
    nch = n // c
    pos_c = lax.broadcasted_iota(jnp.int32, (c, 1), 0)

    d0 = d_ref[...]
    gt_ref[...] = jnp.broadcast_to(jnp.sum(d0 * t_ref[...]), (1, 1, 1))
    vs_ref[...] = d0.T

    def pass0(ci, _):
        base = ci * c
        v = vs_ref[pl.ds(base, c), :]
        idx = lax.broadcasted_iota(jnp.int32, (c, b), 0) + base
        pos = pos_c + base
        size = 2
        while size <= c:
            asc = (pos & size) == 0
            j = size // 2
            while j >= 1:
                v, idx = _stage(v, idx, pos_c, j, asc, c)
                j //= 2
            size *= 2
        vs_ref[pl.ds(base, c), :] = v
        is_ref[pl.ds(base, c), :] = idx
        return 0

    lax.fori_loop(0, nch, pass0, 0)

    pos_f = lax.broadcasted_iota(jnp.int32, (n, 1), 0)
    size = 2 * c
    while size <= n:
        asc_f = (pos_f & size) == 0
        j = size // 2
        while j >= c:
            v = vs_ref[...]
            idx = is_ref[...]
            v, idx = _stage(v, idx, pos_f, j, asc_f, n)
            vs_ref[...] = v
            is_ref[...] = idx
            j //= 2

        def passf(ci, _, size=size):
            base = ci * c
            v = vs_ref[pl.ds(base, c), :]
            idx = is_ref[pl.ds(base, c), :]
            asc = ((pos_c + base) & size) == 0
            j = c // 2
            while j >= 1:
                v, idx = _stage(v, idx, pos_c, j, asc, c)
                j //= 2
            vs_ref[pl.ds(base, c), :] = v
            is_ref[pl.ds(base, c), :] = idx
            return 0

        lax.fori_loop(0, nch, passf, 0)
        size *= 2

    b2 = ki_ref.shape[0]
    k = ki_ref.shape[1] // 2
    r_total = pl.num_programs(0) * b2
    ki_t = is_ref[pl.ds(0, kpad), :][:k]
    kv_t = vs_ref[pl.ds(0, kpad), :][:k]
    rows_t = (lax.broadcasted_iota(jnp.int32, (k, b2), 1)
              + pl.program_id(0) * b2)
    dst_t = ki_t + r_total
    ki_ref[...] = jnp.stack([rows_t, dst_t], axis=1).reshape(2 * k, b2).T
    kr_ref[...] = jnp.stack([dst_t, rows_t], axis=1).reshape(2 * k, b2).T
    kv_ref[...] = jnp.stack([kv_t, kv_t], axis=1).reshape(2 * k, b2).T


def _topk_call(d, t):
    r, n = d.shape
    k = min(r, 10 + 2 * (r // 10))
    kpad = min(_cdiv(k, 8) * 8, n)
    b = min(128, r)
    g = r // b
    gt_p, e0, e1, ea = pl.pallas_call(
        functools.partial(_sort_topk_body, n=n, kpad=kpad),
        grid=(g,),
        in_specs=[
            pl.BlockSpec((b, n), lambda i: (i, 0)),
            pl.BlockSpec((b, n), lambda i: (i, 0)),
        ],
        out_specs=[
            pl.BlockSpec((1, 1, 1), lambda i: (i, 0, 0)),
            pl.BlockSpec((b, 2 * k), lambda i: (i, 0)),
            pl.BlockSpec((b, 2 * k), lambda i: (i, 0)),
            pl.BlockSpec((b, 2 * k), lambda i: (i, 0)),
        ],
        out_shape=[
            jax.ShapeDtypeStruct((g, 1, 1), jnp.float32),
            jax.ShapeDtypeStruct((r, 2 * k), jnp.int32),
            jax.ShapeDtypeStruct((r, 2 * k), jnp.int32),
            jax.ShapeDtypeStruct((r, 2 * k), jnp.float32),
        ],
        scratch_shapes=[
            pltpu.VMEM((n, b), jnp.float32),
            pltpu.VMEM((n, b), jnp.int32),
        ],
        compiler_params=pltpu.CompilerParams(
            dimension_semantics=("parallel",),
            vmem_limit_bytes=67_000_000,
        ),
    )(d, t)
    return gt_p, e0, e1, ea


def kernel(distance_matrix, target):
    r, n = distance_matrix.shape

    gt_p, e0, e1, ea = _topk_call(distance_matrix, target)
    gt = jnp.sum(gt_p)

    edge_index = jnp.stack([e0.reshape(-1), e1.reshape(-1)], axis=0)
    edge_attr = ea.reshape(-1, 1)

    x = jnp.zeros((r + n, 8), dtype=jnp.float32)
    y = target.reshape(-1, 1)
    cost_vec = distance_matrix.reshape(-1, 1)
    return (gt, x, edge_index, edge_attr, y, cost_vec)


# speedup vs baseline: 2.6450x; 1.0007x over previous
import functools

import jax
import jax.numpy as jnp
from jax import lax
from jax.experimental import pallas as pl
from jax.experimental.pallas import tpu as pltpu
from jax.experimental.pallas import tpu_sc as plsc

_SC_LANES = 16
_SC_WORKERS = 32
_SC_BLK = 8


def _gt_cost_sc(d, t):
    r, n = d.shape
    rows_per_w = r // _SC_WORKERS
    n_blocks = rows_per_w // _SC_BLK
    mesh = plsc.VectorSubcoreMesh(core_axis_name="c", subcore_axis_name="s")

    @functools.partial(
        pl.kernel,
        out_type=jax.ShapeDtypeStruct((_SC_WORKERS, _SC_LANES), jnp.float32),
        mesh=mesh,
        scratch_types=[
            pltpu.VMEM((_SC_BLK, n), jnp.float32),
            pltpu.VMEM((_SC_BLK, n), jnp.float32),
            pltpu.VMEM((_SC_LANES,), jnp.float32),
            pltpu.SemaphoreType.DMA,
        ],
    )
    def sc_kernel(d_hbm, t_hbm, o_hbm, db, tb, av, sem):
        wid = lax.axis_index("s") * 2 + lax.axis_index("c")
        base = wid * rows_per_w

        def block(bi, acc):
            row0 = base + bi * _SC_BLK
            pltpu.async_copy(d_hbm.at[pl.ds(row0, _SC_BLK)], db, sem).wait()
            pltpu.async_copy(t_hbm.at[pl.ds(row0, _SC_BLK)], tb, sem).wait()

            def row(i, acc_r):
                def col(ci, acc_c):
                    s = pl.ds(ci * _SC_LANES, _SC_LANES)
                    return acc_c + db[i, s] * tb[i, s]

                return lax.fori_loop(0, n // _SC_LANES, col, acc_r)

            return lax.fori_loop(0, _SC_BLK, row, acc)

        acc = lax.fori_loop(0, n_blocks, block,
                            jnp.zeros((_SC_LANES,), jnp.float32))
        av[...] = acc
        pltpu.sync_copy(av, o_hbm.at[wid])

    return sc_kernel(d, t)

_CHUNK = 128


def _cdiv(a, b):
    return (a + b - 1) // b


def _cmp_exchange(v, idx, vp, ip, low, asc):
    less = (v < vp) | ((v == vp) & (idx < ip))
    sel = less == (low == asc)
    return jnp.where(sel, v, vp), jnp.where(sel, idx, ip)


def _stage(v, idx, pos, j, asc, m):
    low = (pos & j) == 0
    vp = jnp.where(low, pltpu.roll(v, m - j, 0), pltpu.roll(v, j, 0))
    ip = jnp.where(low, pltpu.roll(idx, m - j, 0), pltpu.roll(idx, j, 0))
    return _cmp_exchange(v, idx, vp, ip, low, asc)


def _sort_topk_body(d_ref, ki_ref, kr_ref, kv_ref,
                    vs_ref, is_ref, *, n, kpad):
    b = d_ref.shape[0]
    c = min(_CHUNK, n)
    nch = n // c
    pos_c = lax.broadcasted_iota(jnp.int32, (c, 1), 0)

    vs_ref[...] = d_ref[...].T

    def pass0(ci, _):
        base = ci * c
        v = vs_ref[pl.ds(base, c), :]
        idx = lax.broadcasted_iota(jnp.int32, (c, b), 0) + base
        pos = pos_c + base
        size = 2
        while size <= c:
            asc = (pos & size) == 0
            j = size // 2
            while j >= 1:
                v, idx = _stage(v, idx, pos_c, j, asc, c)
                j //= 2
            size *= 2
        vs_ref[pl.ds(base, c), :] = v
        is_ref[pl.ds(base, c), :] = idx
        return 0

    lax.fori_loop(0, nch, pass0, 0)

    pos_f = lax.broadcasted_iota(jnp.int32, (n, 1), 0)
    size = 2 * c
    while size <= n:
        asc_f = (pos_f & size) == 0
        j = size // 2
        while j >= c:
            v = vs_ref[...]
            idx = is_ref[...]
            v, idx = _stage(v, idx, pos_f, j, asc_f, n)
            vs_ref[...] = v
            is_ref[...] = idx
            j //= 2

        def passf(ci, _, size=size):
            base = ci * c
            v = vs_ref[pl.ds(base, c), :]
            idx = is_ref[pl.ds(base, c), :]
            asc = ((pos_c + base) & size) == 0
            j = c // 2
            while j >= 1:
                v, idx = _stage(v, idx, pos_c, j, asc, c)
                j //= 2
            vs_ref[pl.ds(base, c), :] = v
            is_ref[pl.ds(base, c), :] = idx
            return 0

        lax.fori_loop(0, nch, passf, 0)
        size *= 2

    b2 = ki_ref.shape[0]
    k = ki_ref.shape[1] // 2
    r_total = pl.num_programs(0) * b2
    ki_t = is_ref[pl.ds(0, kpad), :][:k]
    kv_t = vs_ref[pl.ds(0, kpad), :][:k]
    rows_t = (lax.broadcasted_iota(jnp.int32, (k, b2), 1)
              + pl.program_id(0) * b2)
    dst_t = ki_t + r_total
    ki_ref[...] = jnp.stack([rows_t, dst_t], axis=1).reshape(2 * k, b2).T
    kr_ref[...] = jnp.stack([dst_t, rows_t], axis=1).reshape(2 * k, b2).T
    kv_ref[...] = jnp.stack([kv_t, kv_t], axis=1).reshape(2 * k, b2).T


def _topk_call(d):
    r, n = d.shape
    k = min(r, 10 + 2 * (r // 10))
    kpad = min(_cdiv(k, 8) * 8, n)
    b = min(128, r)
    g = r // b
    e0, e1, ea = pl.pallas_call(
        functools.partial(_sort_topk_body, n=n, kpad=kpad),
        grid=(g,),
        in_specs=[
            pl.BlockSpec((b, n), lambda i: (i, 0)),
        ],
        out_specs=[
            pl.BlockSpec((b, 2 * k), lambda i: (i, 0)),
            pl.BlockSpec((b, 2 * k), lambda i: (i, 0)),
            pl.BlockSpec((b, 2 * k), lambda i: (i, 0)),
        ],
        out_shape=[
            jax.ShapeDtypeStruct((r, 2 * k), jnp.int32),
            jax.ShapeDtypeStruct((r, 2 * k), jnp.int32),
            jax.ShapeDtypeStruct((r, 2 * k), jnp.float32),
        ],
        scratch_shapes=[
            pltpu.VMEM((n, b), jnp.float32),
            pltpu.VMEM((n, b), jnp.int32),
        ],
        compiler_params=pltpu.CompilerParams(
            dimension_semantics=("parallel",),
            vmem_limit_bytes=67_000_000,
        ),
    )(d)
    return e0, e1, ea


def kernel(distance_matrix, target):
    r, n = distance_matrix.shape

    e0, e1, ea = _topk_call(distance_matrix)
    gt = jnp.sum(_gt_cost_sc(distance_matrix, target))

    edge_index = jnp.stack([e0.reshape(-1), e1.reshape(-1)], axis=0)
    edge_attr = ea.reshape(-1, 1)

    x = jnp.zeros((r + n, 8), dtype=jnp.float32)
    y = target.reshape(-1, 1)
    cost_vec = distance_matrix.reshape(-1, 1)
    return (gt, x, edge_index, edge_attr, y, cost_vec)


# speedup vs baseline: 2.7730x; 1.0484x over previous
import functools

import jax
import jax.numpy as jnp
from jax import lax
from jax.experimental import pallas as pl
from jax.experimental.pallas import tpu as pltpu
from jax.experimental.pallas import tpu_sc as plsc

_SC_LANES = 16
_SC_WORKERS = 32
_SC_BLK = 8


def _gt_cost_sc(d, t):
    r, n = d.shape
    rows_per_w = r // _SC_WORKERS
    n_blocks = rows_per_w // _SC_BLK
    mesh = plsc.VectorSubcoreMesh(core_axis_name="c", subcore_axis_name="s")

    @functools.partial(
        pl.kernel,
        out_type=jax.ShapeDtypeStruct((_SC_WORKERS, _SC_LANES), jnp.float32),
        mesh=mesh,
        scratch_types=[
            pltpu.VMEM((_SC_BLK, n), jnp.float32),
            pltpu.VMEM((_SC_BLK, n), jnp.float32),
            pltpu.VMEM((_SC_LANES,), jnp.float32),
            pltpu.SemaphoreType.DMA,
        ],
    )
    def sc_kernel(d_hbm, t_hbm, o_hbm, db, tb, av, sem):
        wid = lax.axis_index("s") * 2 + lax.axis_index("c")
        base = wid * rows_per_w

        def block(bi, acc):
            row0 = base + bi * _SC_BLK
            pltpu.async_copy(d_hbm.at[pl.ds(row0, _SC_BLK)], db, sem).wait()
            pltpu.async_copy(t_hbm.at[pl.ds(row0, _SC_BLK)], tb, sem).wait()

            def row(i, acc_r):
                def col(ci, acc_c):
                    s = pl.ds(ci * _SC_LANES, _SC_LANES)
                    return acc_c + db[i, s] * tb[i, s]

                return lax.fori_loop(0, n // _SC_LANES, col, acc_r)

            return lax.fori_loop(0, _SC_BLK, row, acc)

        acc = lax.fori_loop(0, n_blocks, block,
                            jnp.zeros((_SC_LANES,), jnp.float32))
        av[...] = acc
        pltpu.sync_copy(av, o_hbm.at[wid])

    return sc_kernel(d, t)

_CHUNK = 128


def _cdiv(a, b):
    return (a + b - 1) // b


def _cmp_exchange(v, idx, vp, ip, low, asc):
    less = (v < vp) | ((v == vp) & (idx < ip))
    sel = less == (low == asc)
    return jnp.where(sel, v, vp), jnp.where(sel, idx, ip)


def _stage(v, idx, pos, j, asc, m):
    low = (pos & j) == 0
    vp = jnp.where(low, pltpu.roll(v, m - j, 0), pltpu.roll(v, j, 0))
    ip = jnp.where(low, pltpu.roll(idx, m - j, 0), pltpu.roll(idx, j, 0))
    return _cmp_exchange(v, idx, vp, ip, low, asc)


def _sort_topk_body(d_ref, ki_ref, kr_ref, kv_ref,
                    vs_ref, is_ref, *, n, kpad):
    b = d_ref.shape[0]
    c = min(_CHUNK, n)
    nch = n // c
    pos_c = lax.broadcasted_iota(jnp.int32, (c, 1), 0)

    vs_ref[...] = d_ref[...].T

    def pass0(ci, _):
        base = ci * c
        v = vs_ref[pl.ds(base, c), :]
        idx = lax.broadcasted_iota(jnp.int32, (c, b), 0) + base
        pos = pos_c + base
        size = 2
        while size <= c:
            asc = (pos & size) == 0
            j = size // 2
            while j >= 1:
                v, idx = _stage(v, idx, pos_c, j, asc, c)
                j //= 2
            size *= 2
        vs_ref[pl.ds(base, c), :] = v
        is_ref[pl.ds(base, c), :] = idx
        return 0

    lax.fori_loop(0, nch, pass0, 0)

    pos_f = lax.broadcasted_iota(jnp.int32, (n, 1), 0)
    size = 2 * c
    while size <= n:
        asc_f = (pos_f & size) == 0
        j = size // 2
        while j >= c:
            v = vs_ref[...]
            idx = is_ref[...]
            v, idx = _stage(v, idx, pos_f, j, asc_f, n)
            vs_ref[...] = v
            is_ref[...] = idx
            j //= 2

        def passf(ci, _, size=size):
            base = ci * c
            v = vs_ref[pl.ds(base, c), :]
            idx = is_ref[pl.ds(base, c), :]
            asc = ((pos_c + base) & size) == 0
            j = c // 2
            while j >= 1:
                v, idx = _stage(v, idx, pos_c, j, asc, c)
                j //= 2
            vs_ref[pl.ds(base, c), :] = v
            is_ref[pl.ds(base, c), :] = idx
            return 0

        lax.fori_loop(0, nch, passf, 0)
        size *= 2

    b2 = ki_ref.shape[0]
    k = ki_ref.shape[1] // 2
    r_total = pl.num_programs(0) * b2
    ki_t = is_ref[pl.ds(0, kpad), :][:k]
    kv_t = vs_ref[pl.ds(0, kpad), :][:k]
    rows_t = (lax.broadcasted_iota(jnp.int32, (k, b2), 1)
              + pl.program_id(0) * b2)
    dst_t = ki_t + r_total
    ki_ref[...] = jnp.stack([rows_t, dst_t], axis=1).reshape(2 * k, b2).T
    kr_ref[...] = jnp.stack([dst_t, rows_t], axis=1).reshape(2 * k, b2).T
    kv_ref[...] = jnp.stack([kv_t, kv_t], axis=1).reshape(2 * k, b2).T


def _topk_call(d):
    r, n = d.shape
    k = min(r, 10 + 2 * (r // 10))
    kpad = min(_cdiv(k, 8) * 8, n)
    b = min(256, r)
    g = r // b
    e0, e1, ea = pl.pallas_call(
        functools.partial(_sort_topk_body, n=n, kpad=kpad),
        grid=(g,),
        in_specs=[
            pl.BlockSpec((b, n), lambda i: (i, 0)),
        ],
        out_specs=[
            pl.BlockSpec((b, 2 * k), lambda i: (i, 0)),
            pl.BlockSpec((b, 2 * k), lambda i: (i, 0)),
            pl.BlockSpec((b, 2 * k), lambda i: (i, 0)),
        ],
        out_shape=[
            jax.ShapeDtypeStruct((r, 2 * k), jnp.int32),
            jax.ShapeDtypeStruct((r, 2 * k), jnp.int32),
            jax.ShapeDtypeStruct((r, 2 * k), jnp.float32),
        ],
        scratch_shapes=[
            pltpu.VMEM((n, b), jnp.float32),
            pltpu.VMEM((n, b), jnp.int32),
        ],
        compiler_params=pltpu.CompilerParams(
            dimension_semantics=("parallel",),
            vmem_limit_bytes=67_000_000,
        ),
    )(d)
    return e0, e1, ea


def kernel(distance_matrix, target):
    r, n = distance_matrix.shape

    e0, e1, ea = _topk_call(distance_matrix)
    gt = jnp.sum(_gt_cost_sc(distance_matrix, target))

    edge_index = jnp.stack([e0.reshape(-1), e1.reshape(-1)], axis=0)
    edge_attr = ea.reshape(-1, 1)

    x = jnp.zeros((r + n, 8), dtype=jnp.float32)
    y = target.reshape(-1, 1)
    cost_vec = distance_matrix.reshape(-1, 1)
    return (gt, x, edge_index, edge_attr, y, cost_vec)


# speedup vs baseline: 2.8794x; 1.0384x over previous
import functools

import jax
import jax.numpy as jnp
from jax import lax
from jax.experimental import pallas as pl
from jax.experimental.pallas import tpu as pltpu
from jax.experimental.pallas import tpu_sc as plsc

_SC_LANES = 16
_SC_WORKERS = 32
_SC_BLK = 8


def _gt_cost_sc(d, t):
    r, n = d.shape
    rows_per_w = r // _SC_WORKERS
    n_blocks = rows_per_w // _SC_BLK
    mesh = plsc.VectorSubcoreMesh(core_axis_name="c", subcore_axis_name="s")

    @functools.partial(
        pl.kernel,
        out_type=jax.ShapeDtypeStruct((_SC_WORKERS, _SC_LANES), jnp.float32),
        mesh=mesh,
        scratch_types=[
            pltpu.VMEM((_SC_BLK, n), jnp.float32),
            pltpu.VMEM((_SC_BLK, n), jnp.float32),
            pltpu.VMEM((_SC_LANES,), jnp.float32),
            pltpu.SemaphoreType.DMA,
        ],
    )
    def sc_kernel(d_hbm, t_hbm, o_hbm, db, tb, av, sem):
        wid = lax.axis_index("s") * 2 + lax.axis_index("c")
        base = wid * rows_per_w

        def block(bi, acc):
            row0 = base + bi * _SC_BLK
            pltpu.async_copy(d_hbm.at[pl.ds(row0, _SC_BLK)], db, sem).wait()
            pltpu.async_copy(t_hbm.at[pl.ds(row0, _SC_BLK)], tb, sem).wait()

            def row(i, acc_r):
                def col(ci, acc_c):
                    s = pl.ds(ci * _SC_LANES, _SC_LANES)
                    return acc_c + db[i, s] * tb[i, s]

                return lax.fori_loop(0, n // _SC_LANES, col, acc_r)

            return lax.fori_loop(0, _SC_BLK, row, acc)

        acc = lax.fori_loop(0, n_blocks, block,
                            jnp.zeros((_SC_LANES,), jnp.float32))
        av[...] = acc
        pltpu.sync_copy(av, o_hbm.at[wid])

    return sc_kernel(d, t)

_CHUNK = 128


def _cdiv(a, b):
    return (a + b - 1) // b


def _cmp_exchange(v, idx, vp, ip, low, asc):
    less = (v < vp) | ((v == vp) & (idx < ip))
    sel = less == (low == asc)
    return jnp.where(sel, v, vp), jnp.where(sel, idx, ip)


def _stage(v, idx, pos, j, asc, m):
    low = (pos & j) == 0
    vp = jnp.where(low, pltpu.roll(v, m - j, 0), pltpu.roll(v, j, 0))
    ip = jnp.where(low, pltpu.roll(idx, m - j, 0), pltpu.roll(idx, j, 0))
    return _cmp_exchange(v, idx, vp, ip, low, asc)


def _sort_topk_body(d_ref, ki_ref, kr_ref, kv_ref,
                    vs_ref, is_ref, *, n, kpad):
    b = d_ref.shape[0]
    c = min(_CHUNK, n)
    nch = n // c
    pos_c = lax.broadcasted_iota(jnp.int32, (c, 1), 0)

    vs_ref[...] = d_ref[...].T

    def pass0(ci, _):
        base = ci * c
        v = vs_ref[pl.ds(base, c), :]
        idx = lax.broadcasted_iota(jnp.int32, (c, b), 0) + base
        pos = pos_c + base
        size = 2
        while size <= c:
            asc = (pos & size) == 0
            j = size // 2
            while j >= 1:
                v, idx = _stage(v, idx, pos_c, j, asc, c)
                j //= 2
            size *= 2
        vs_ref[pl.ds(base, c), :] = v
        is_ref[pl.ds(base, c), :] = idx
        return 0

    lax.fori_loop(0, nch, pass0, 0)

    pos_f = lax.broadcasted_iota(jnp.int32, (n, 1), 0)
    size = 2 * c
    while size <= n:
        asc_f = (pos_f & size) == 0
        j = size // 2
        while j >= c:
            v = vs_ref[...]
            idx = is_ref[...]
            v, idx = _stage(v, idx, pos_f, j, asc_f, n)
            vs_ref[...] = v
            is_ref[...] = idx
            j //= 2

        def passf(ci, _, size=size):
            base = ci * c
            v = vs_ref[pl.ds(base, c), :]
            idx = is_ref[pl.ds(base, c), :]
            asc = ((pos_c + base) & size) == 0
            j = c // 2
            while j >= 1:
                v, idx = _stage(v, idx, pos_c, j, asc, c)
                j //= 2
            vs_ref[pl.ds(base, c), :] = v
            is_ref[pl.ds(base, c), :] = idx
            return 0

        lax.fori_loop(0, nch, passf, 0)
        size *= 2

    b2 = ki_ref.shape[0]
    k = ki_ref.shape[1] // 2
    r_total = pl.num_programs(0) * b2
    ki_t = is_ref[pl.ds(0, kpad), :][:k]
    kv_t = vs_ref[pl.ds(0, kpad), :][:k]
    rows_t = (lax.broadcasted_iota(jnp.int32, (k, b2), 1)
              + pl.program_id(0) * b2)
    dst_t = ki_t + r_total
    ki_ref[...] = jnp.stack([rows_t, dst_t], axis=1).reshape(2 * k, b2).T
    kr_ref[...] = jnp.stack([dst_t, rows_t], axis=1).reshape(2 * k, b2).T
    kv_ref[...] = jnp.stack([kv_t, kv_t], axis=1).reshape(2 * k, b2).T


def _topk_call(d):
    r, n = d.shape
    k = min(r, 10 + 2 * (r // 10))
    kpad = min(_cdiv(k, 8) * 8, n)
    b = min(512, r)
    g = r // b
    e0, e1, ea = pl.pallas_call(
        functools.partial(_sort_topk_body, n=n, kpad=kpad),
        grid=(g,),
        in_specs=[
            pl.BlockSpec((b, n), lambda i: (i, 0)),
        ],
        out_specs=[
            pl.BlockSpec((b, 2 * k), lambda i: (i, 0)),
            pl.BlockSpec((b, 2 * k), lambda i: (i, 0)),
            pl.BlockSpec((b, 2 * k), lambda i: (i, 0)),
        ],
        out_shape=[
            jax.ShapeDtypeStruct((r, 2 * k), jnp.int32),
            jax.ShapeDtypeStruct((r, 2 * k), jnp.int32),
            jax.ShapeDtypeStruct((r, 2 * k), jnp.float32),
        ],
        scratch_shapes=[
            pltpu.VMEM((n, b), jnp.float32),
            pltpu.VMEM((n, b), jnp.int32),
        ],
        compiler_params=pltpu.CompilerParams(
            dimension_semantics=("parallel",),
            vmem_limit_bytes=67_000_000,
        ),
    )(d)
    return e0, e1, ea


def kernel(distance_matrix, target):
    r, n = distance_matrix.shape

    e0, e1, ea = _topk_call(distance_matrix)
    gt = jnp.sum(_gt_cost_sc(distance_matrix, target))

    edge_index = jnp.stack([e0.reshape(-1), e1.reshape(-1)], axis=0)
    edge_attr = ea.reshape(-1, 1)

    x = jnp.zeros((r + n, 8), dtype=jnp.float32)
    y = target.reshape(-1, 1)
    cost_vec = distance_matrix.reshape(-1, 1)
    return (gt, x, edge_index, edge_attr, y, cost_vec)
